# baseline (speedup 1.0000x reference)
import functools
import math

import jax
import jax.numpy as jnp
from jax import lax
from jax.experimental import pallas as pl
from jax.experimental.pallas import tpu as pltpu
from jax.experimental.pallas import tpu_sc as plsc

NC = 2
NS = 16
NW = NC * NS
LANES = 16

D = 64
BATCH = 4096
HIST = 200
NB = BATCH // 128
ND = D // 8
SCALE = math.sqrt(float(D))

_mesh = plsc.VectorSubcoreMesh(
    core_axis_name="c", subcore_axis_name="s", num_cores=NC, num_subcores=NS
)


@functools.partial(
    pl.kernel,
    out_type=jax.ShapeDtypeStruct((HIST, ND, NB, 8, 128), jnp.float32),
    mesh=_mesh,
    scratch_types=[
        pltpu.VMEM((128,), jnp.int32),
        pltpu.VMEM((128,), jnp.int32),
        pltpu.VMEM((128, 128), jnp.float32),
        pltpu.VMEM((8, 128), jnp.float32),
        pltpu.SemaphoreType.DMA,
    ],
    compiler_params=pltpu.CompilerParams(
        use_tc_tiling_on_sc=False, needs_layout_passes=False
    ),
)
def _emb_lookup(table_hbm, srct_hbm, out_hbm, idx_v, pidx_v, rows_v, tile_v, sem):
    wid = lax.axis_index("s") * NC + lax.axis_index("c")

    iotas = [lax.iota(jnp.int32, LANES) + (l0 * LANES) for l0 in range(8)]

    @pl.loop(0, HIST)
    def _step(h):
        pltpu.sync_copy(srct_hbm.at[h, pl.ds(wid * 128, 128)], idx_v)
        pars = []
        for l0 in range(8):
            sl = pl.ds(l0 * LANES, LANES)
            w = idx_v[sl]
            pidx_v[sl] = lax.shift_right_logical(w, 1)
            pars.append(lax.bitwise_and(w, 1) * D)
        pltpu.async_copy(table_hbm.at[pidx_v], rows_v, sem).wait()
        for td in range(ND):
            for s in range(8):
                base = jnp.int32(td * 8 + s)
                for l0 in range(8):
                    v = plsc.load_gather(rows_v, [iotas[l0], pars[l0] + base])
                    tile_v[s, pl.ds(l0 * LANES, LANES)] = v * SCALE
            pltpu.sync_copy(tile_v, out_hbm.at[h, td, wid])


def kernel(src, emb_weight):
    src_t = src.T.astype(jnp.int32)
    table2 = emb_weight.reshape(500000, 128)
    x = _emb_lookup(table2, src_t)
    out = jnp.transpose(x, (2, 4, 0, 1, 3))
    return out.reshape(BATCH, HIST, D)


# speedup vs baseline: 1.4058x; 1.4058x over previous
import functools
import math

import jax
import jax.numpy as jnp
from jax import lax
from jax.experimental import pallas as pl
from jax.experimental.pallas import tpu as pltpu
from jax.experimental.pallas import tpu_sc as plsc

NC = 2
NS = 16
NW = NC * NS
LANES = 16

D = 64
NTOK = 1000000
BATCH = 4096
HIST = 200
NB = BATCH // 128
ND = D // 8
SCALE = math.sqrt(float(D))

_mesh = plsc.VectorSubcoreMesh(
    core_axis_name="c", subcore_axis_name="s", num_cores=NC, num_subcores=NS
)


@functools.partial(
    pl.kernel,
    out_type=jax.ShapeDtypeStruct((HIST, ND, NB, 8, 128), jnp.float32),
    mesh=_mesh,
    scratch_types=[
        pltpu.VMEM((128,), jnp.int32),
        pltpu.VMEM((128,), jnp.int32),
        pltpu.VMEM((128,), jnp.int32),
        pltpu.VMEM((128,), jnp.int32),
        pltpu.VMEM((128, D), jnp.float32),
        pltpu.VMEM((128, D), jnp.float32),
        pltpu.VMEM((ND, 8, 128), jnp.float32),
        pltpu.SemaphoreType.DMA,
        pltpu.SemaphoreType.DMA,
    ],
    compiler_params=pltpu.CompilerParams(
        use_tc_tiling_on_sc=False, needs_layout_passes=False
    ),
)
def _emb_lookup(
    table_hbm, srct_hbm, out_hbm,
    idx_a, idx_b, pidx_a, pidx_b, rows_a, rows_b, tiles_v, sem_a, sem_b,
):
    wid = lax.axis_index("s") * NC + lax.axis_index("c")
    col0 = wid * 128

    def stage(h, idx_v, pidx_v, rows_v, sem):
        pltpu.sync_copy(srct_hbm.at[h, pl.ds(col0, 128)], idx_v)
        for o in range(0, 128, LANES):
            sl = pl.ds(o, LANES)
            pidx_v[sl] = idx_v[sl] * 2
        pltpu.async_copy(table_hbm.at[pidx_v], rows_v, sem)

    def gather_wait(pidx_v, rows_v, sem):
        pltpu.make_async_copy(table_hbm.at[pidx_v], rows_v, sem).wait()

    def consume(h, rows_v):
        @plsc.parallel_loop(0, 512, unroll=4)
        def _t(i):
            l0 = lax.bitwise_and(i, 7)
            c = lax.shift_right_logical(i, 3)
            td = lax.shift_right_logical(i, 6)
            s = lax.bitwise_and(c, 7)
            rid = lax.iota(jnp.int32, LANES) + l0 * LANES
            cid = jnp.zeros((LANES,), jnp.int32) + c
            v = plsc.load_gather(rows_v, [rid, cid])
            tiles_v[td, s, pl.ds(l0 * LANES, LANES)] = v

        pltpu.sync_copy(tiles_v, out_hbm.at[h, :, wid])

    stage(0, idx_a, pidx_a, rows_a, sem_a)

    @pl.loop(0, HIST, step=2)
    def _step(h):
        stage(h + 1, idx_b, pidx_b, rows_b, sem_b)
        gather_wait(pidx_a, rows_a, sem_a)
        consume(h, rows_a)

        @pl.when(h + 2 < HIST)
        def _prefetch():
            stage(h + 2, idx_a, pidx_a, rows_a, sem_a)

        gather_wait(pidx_b, rows_b, sem_b)
        consume(h + 1, rows_b)


def kernel(src, emb_weight):
    src_t = src.T.astype(jnp.int32)
    table3 = jnp.pad(emb_weight * SCALE, ((0, 0), (0, D))).reshape(2 * NTOK, D)
    x = _emb_lookup(table3, src_t)
    out = jnp.transpose(x, (2, 4, 0, 1, 3))
    return out.reshape(BATCH, HIST, D)
